# baseline (speedup 1.0000x reference)
import functools

import jax
import jax.numpy as jnp
from jax import lax
from jax.experimental import pallas as pl
from jax.experimental.pallas import tpu as pltpu
from jax.experimental.pallas import tpu_sc as plsc

N = 10000
E = 320000
D = 128
DE = 16
L = 5
G = 128
T = 12

NC, NS = 2, 16
NW = NC * NS
C = 80
CHUNKS_PER_W = E // (NW * C)
ZR = 16
WB = 208


def _edge_embed_body(ea_ref, w_ref, out_ref):
    out_ref[...] = jnp.maximum(
        jnp.dot(ea_ref[...], w_ref[...], preferred_element_type=jnp.float32), 0.0)


def _edge_embed(edge_attr, w):
    EB = 8000
    return pl.pallas_call(
        _edge_embed_body,
        grid=(E // EB,),
        in_specs=[pl.BlockSpec((EB, DE), lambda i: (i, 0)),
                  pl.BlockSpec((DE, D), lambda i: (0, 0))],
        out_specs=pl.BlockSpec((EB, D), lambda i: (i, 0)),
        out_shape=jax.ShapeDtypeStruct((E, D), jnp.float32),
    )(edge_attr, w)


def _sc_aggr_body(h_hbm, e_hbm, src_hbm, dst_hbm, out_hbm,
                  src_a, dst_a, msg_a, gth_a, src_b, dst_b, msg_b, gth_b,
                  zero_v, aggr_sh, sem_ia, sem_ib, sem_ga, sem_gb,
                  sem_sa, sem_sb):
    c = lax.axis_index("c")
    s = lax.axis_index("s")
    w = s * NC + c

    def zrow(r, carry):
        for q in range(D // 16):
            zero_v[r, pl.ds(q * 16, 16)] = jnp.zeros((16,), jnp.float32)
        return carry
    lax.fori_loop(0, ZR, zrow, 0)

    def zcopy(k, carry):
        pltpu.sync_copy(zero_v, aggr_sh.at[pl.ds(s * 624 + k * ZR, ZR), :])
        return carry
    lax.fori_loop(0, 624 // ZR, zcopy, 0)

    @pl.when(s == NS - 1)
    def _ztail():
        pltpu.sync_copy(zero_v, aggr_sh.at[pl.ds(N - 16, 16), :])
    plsc.subcore_barrier()

    def issue_in(t, src_v, dst_v, msg_v, sem):
        base = (w * CHUNKS_PER_W + t) * C
        pltpu.async_copy(src_hbm.at[pl.ds(base, C)], src_v, sem)
        pltpu.async_copy(dst_hbm.at[pl.ds(base, C)], dst_v, sem)
        pltpu.async_copy(e_hbm.at[pl.ds(base, C), :], msg_v, sem)

    def wait_in(t, src_v, dst_v, msg_v, sem):
        base = (w * CHUNKS_PER_W + t) * C
        pltpu.make_async_copy(src_hbm.at[pl.ds(base, C)], src_v, sem).wait()
        pltpu.make_async_copy(dst_hbm.at[pl.ds(base, C)], dst_v, sem).wait()
        pltpu.make_async_copy(e_hbm.at[pl.ds(base, C), :], msg_v, sem).wait()

    def issue_g(src_v, gth_v, sem):
        return pltpu.async_copy(h_hbm.at[src_v], gth_v, sem)

    def wait_g(src_v, gth_v, sem):
        pltpu.make_async_copy(h_hbm.at[src_v], gth_v, sem).wait()

    def relu(msg_v, gth_v):
        @plsc.parallel_loop(0, C, unroll=4)
        def _row(r):
            for q in range(D // 16):
                sl = pl.ds(q * 16, 16)
                msg_v[r, sl] = jnp.maximum(msg_v[r, sl] + gth_v[r, sl], 0.0)

    bufa = (src_a, dst_a, msg_a, sem_ia)
    bufb = (src_b, dst_b, msg_b, sem_ib)

    issue_in(0, *bufa)
    wait_in(0, *bufa)
    issue_g(src_a, gth_a, sem_ga)
    issue_in(1, *bufb)

    def pair(i, carry):
        t = 2 * i
        wait_g(src_a, gth_a, sem_ga)
        wait_in(t + 1, *bufb)
        issue_g(src_b, gth_b, sem_gb)
        relu(msg_a, gth_a)
        sc_a = pltpu.async_copy(msg_a, aggr_sh.at[dst_a], sem_sa, add=True)
        wait_g(src_b, gth_b, sem_gb)
        sc_a.wait()
        issue_in(t + 2, *bufa)
        relu(msg_b, gth_b)
        sc_b = pltpu.async_copy(msg_b, aggr_sh.at[dst_b], sem_sb, add=True)
        wait_in(t + 2, *bufa)
        issue_g(src_a, gth_a, sem_ga)
        sc_b.wait()

        @pl.when(t + 3 < CHUNKS_PER_W)
        def _pre():
            issue_in(t + 3, *bufb)
        return carry
    lax.fori_loop(0, (CHUNKS_PER_W - 1) // 2, pair, 0)

    wait_g(src_a, gth_a, sem_ga)
    relu(msg_a, gth_a)
    pltpu.sync_copy(msg_a, aggr_sh.at[dst_a], add=True)

    plsc.subcore_barrier()
    for k in range(3):
        pltpu.sync_copy(aggr_sh.at[pl.ds(s * 624 + k * WB, WB), :],
                        out_hbm.at[c, pl.ds(s * 624 + k * WB, WB), :])

    @pl.when(s == NS - 1)
    def _otail():
        pltpu.sync_copy(aggr_sh.at[pl.ds(N - 16, 16), :],
                        out_hbm.at[c, pl.ds(N - 16, 16), :])


_sc_aggregate = pl.kernel(
    _sc_aggr_body,
    out_type=jax.ShapeDtypeStruct((NC, N, D), jnp.float32),
    mesh=plsc.VectorSubcoreMesh(core_axis_name="c", subcore_axis_name="s",
                                num_cores=NC, num_subcores=NS),
    scratch_types=[
        pltpu.VMEM((C,), jnp.int32),
        pltpu.VMEM((C,), jnp.int32),
        pltpu.VMEM((C, D), jnp.float32),
        pltpu.VMEM((C, D), jnp.float32),
        pltpu.VMEM((C,), jnp.int32),
        pltpu.VMEM((C,), jnp.int32),
        pltpu.VMEM((C, D), jnp.float32),
        pltpu.VMEM((C, D), jnp.float32),
        pltpu.VMEM((ZR, D), jnp.float32),
        pltpu.VMEM_SHARED((N, D), jnp.float32),
        pltpu.SemaphoreType.DMA,
        pltpu.SemaphoreType.DMA,
        pltpu.SemaphoreType.DMA,
        pltpu.SemaphoreType.DMA,
        pltpu.SemaphoreType.DMA,
        pltpu.SemaphoreType.DMA,
    ],
)


def _mlp_body(h_ref, a_ref, w1_ref, b1_ref, w2_ref, b2_ref, sc_ref, out_ref,
              *, last):
    z = h_ref[...] * sc_ref[0, 0] + a_ref[0] + a_ref[1]
    z = jnp.maximum(
        jnp.dot(z, w1_ref[...], preferred_element_type=jnp.float32)
        + b1_ref[...], 0.0)
    z = jnp.dot(z, w2_ref[...], preferred_element_type=jnp.float32) + b2_ref[...]
    out_ref[...] = z if last else jnp.maximum(z, 0.0)


def _mlp(h, aggr, w1, b1, w2, b2, scale, last):
    NB = 1000
    return pl.pallas_call(
        functools.partial(_mlp_body, last=last),
        grid=(N // NB,),
        in_specs=[pl.BlockSpec((NB, D), lambda i: (i, 0)),
                  pl.BlockSpec((NC, NB, D), lambda i: (0, i, 0)),
                  pl.BlockSpec((D, D), lambda i: (0, 0)),
                  pl.BlockSpec((1, D), lambda i: (0, 0)),
                  pl.BlockSpec((D, D), lambda i: (0, 0)),
                  pl.BlockSpec((1, D), lambda i: (0, 0)),
                  pl.BlockSpec((1, 1), lambda i: (0, 0))],
        out_specs=pl.BlockSpec((NB, D), lambda i: (i, 0)),
        out_shape=jax.ShapeDtypeStruct((N, D), jnp.float32),
    )(h, aggr, w1, b1, w2, b2, scale)


def _pool_body(b_ref, h_ref, wp_ref, bp_ref, out_ref, pooled_acc, cnt_acc):
    i = pl.program_id(0)
    nblk = pl.num_programs(0)

    @pl.when(i == 0)
    def _init():
        pooled_acc[...] = jnp.zeros_like(pooled_acc)
        cnt_acc[...] = jnp.zeros_like(cnt_acc)

    nb = b_ref.shape[2]
    b = b_ref[0, 0, :]
    iota = lax.broadcasted_iota(jnp.int32, (nb, G), 1)
    p = (b[:, None] == iota).astype(jnp.float32)
    pooled_acc[...] += lax.dot_general(
        p, h_ref[...], (((0,), (0,)), ((), ())),
        preferred_element_type=jnp.float32)
    cnt_acc[...] += lax.dot_general(
        p, jnp.ones((nb, D), jnp.float32), (((0,), (0,)), ((), ())),
        preferred_element_type=jnp.float32)

    @pl.when(i == nblk - 1)
    def _fin():
        pooled = pooled_acc[...] / jnp.maximum(cnt_acc[...], 1.0)
        out_ref[...] = (jnp.dot(pooled, wp_ref[...],
                                preferred_element_type=jnp.float32)
                        + bp_ref[...])


def _pool_head(batch, h, wp, bp):
    NB = 1000
    return pl.pallas_call(
        _pool_body,
        grid=(N // NB,),
        in_specs=[pl.BlockSpec((1, 1, NB), lambda i: (i, 0, 0)),
                  pl.BlockSpec((NB, D), lambda i: (i, 0)),
                  pl.BlockSpec((D, T), lambda i: (0, 0)),
                  pl.BlockSpec((1, T), lambda i: (0, 0))],
        out_specs=pl.BlockSpec((G, T), lambda i: (0, 0)),
        out_shape=jax.ShapeDtypeStruct((G, T), jnp.float32),
        scratch_shapes=[pltpu.VMEM((G, D), jnp.float32),
                        pltpu.VMEM((G, D), jnp.float32)],
    )(batch.reshape(N // NB, 1, NB), h, wp, bp.reshape(1, T))


def kernel(x, edge_index, edge_attr, batch, We, W1, b1, W2, b2, eps, Wp, bp):
    src = edge_index[0]
    dst = edge_index[1]
    h = x
    for l in range(L):
        e = _edge_embed(edge_attr, We[l])
        aggr = _sc_aggregate(h, e, src, dst)
        h = _mlp(h, aggr, W1[l], b1[l].reshape(1, D), W2[l],
                 b2[l].reshape(1, D), (1.0 + eps[l]).reshape(1, 1),
                 last=(l == L - 1))
    return _pool_head(batch, h, Wp, bp)


# speedup vs baseline: 1.0042x; 1.0042x over previous
import functools

import jax
import jax.numpy as jnp
from jax import lax
from jax.experimental import pallas as pl
from jax.experimental.pallas import tpu as pltpu
from jax.experimental.pallas import tpu_sc as plsc

N = 10000
E = 320000
D = 128
DE = 16
L = 5
G = 128
T = 12

NC, NS = 2, 16
NW = NC * NS
C = 80
CHUNKS_PER_W = E // (NW * C)
ZR = 16
WB = 208


def _pack_rows(v):
    lo = v[:, :D // 2].astype(jnp.bfloat16).astype(jnp.float32)
    hi = v[:, D // 2:].astype(jnp.bfloat16).astype(jnp.float32)
    lo_u = jax.lax.bitcast_convert_type(lo, jnp.uint32) >> 16
    hi_u = jax.lax.bitcast_convert_type(hi, jnp.uint32) & jnp.uint32(0xFFFF0000)
    return jax.lax.bitcast_convert_type(lo_u | hi_u, jnp.int32)


def _edge_embed_body(ea_ref, w_ref, out_ref):
    e = jnp.maximum(
        jnp.dot(ea_ref[...], w_ref[...], preferred_element_type=jnp.float32), 0.0)
    out_ref[...] = _pack_rows(e)


def _edge_embed(edge_attr, w):
    EB = 8000
    return pl.pallas_call(
        _edge_embed_body,
        grid=(E // EB,),
        in_specs=[pl.BlockSpec((EB, DE), lambda i: (i, 0)),
                  pl.BlockSpec((DE, D), lambda i: (0, 0))],
        out_specs=pl.BlockSpec((EB, D // 2), lambda i: (i, 0)),
        out_shape=jax.ShapeDtypeStruct((E, D // 2), jnp.int32),
    )(edge_attr, w)


def _sc_aggr_body(h_hbm, e_hbm, src_hbm, dst_hbm, out_hbm,
                  src_a, dst_a, ebf_a, gth_a,
                  src_b, dst_b, ebf_b, gth_b,
                  aggr_sh, sem_ia, sem_ib, sem_ga, sem_gb,
                  sem_sa, sem_sb):
    c = lax.axis_index("c")
    s = lax.axis_index("s")
    w = s * NC + c

    def zrow(r, carry):
        for q in range(D // 16):
            gth_a[r, pl.ds(q * 16, 16)] = jnp.zeros((16,), jnp.float32)
        return carry
    lax.fori_loop(0, ZR, zrow, 0)

    def zcopy(k, carry):
        pltpu.sync_copy(gth_a.at[pl.ds(0, ZR), :],
                        aggr_sh.at[pl.ds(s * 624 + k * ZR, ZR), :])
        return carry
    lax.fori_loop(0, 624 // ZR, zcopy, 0)

    @pl.when(s == NS - 1)
    def _ztail():
        pltpu.sync_copy(gth_a.at[pl.ds(0, ZR), :],
                        aggr_sh.at[pl.ds(N - 16, 16), :])
    plsc.subcore_barrier()

    def issue_in(t, src_v, dst_v, ebf_v, sem):
        base = (w * CHUNKS_PER_W + t) * C
        pltpu.async_copy(src_hbm.at[pl.ds(base, C)], src_v, sem)
        pltpu.async_copy(dst_hbm.at[pl.ds(base, C)], dst_v, sem)
        pltpu.async_copy(e_hbm.at[pl.ds(base, C), :], ebf_v, sem)

    def wait_in(t, src_v, dst_v, ebf_v, sem):
        base = (w * CHUNKS_PER_W + t) * C
        pltpu.make_async_copy(src_hbm.at[pl.ds(base, C)], src_v, sem).wait()
        pltpu.make_async_copy(dst_hbm.at[pl.ds(base, C)], dst_v, sem).wait()
        pltpu.make_async_copy(e_hbm.at[pl.ds(base, C), :], ebf_v, sem).wait()

    def issue_g(src_v, gth_v, sem):
        return pltpu.async_copy(h_hbm.at[src_v], gth_v, sem)

    def wait_g(src_v, gth_v, sem):
        pltpu.make_async_copy(h_hbm.at[src_v], gth_v, sem).wait()

    def _unlo(wd):
        return jax.lax.bitcast_convert_type(wd << 16, jnp.float32)

    def _unhi(wd):
        return jax.lax.bitcast_convert_type(wd & jnp.int32(-65536), jnp.float32)

    def relu(gth_v, ebf_v):
        @plsc.parallel_loop(0, C, unroll=4)
        def _row(r):
            for k in range(D // 32):
                sl = pl.ds(k * 16, 16)
                sh = pl.ds((k + 4) * 16, 16)
                we = ebf_v[r, pl.ds(k * 16, 16)]
                gth_v[r, sl] = jnp.maximum(gth_v[r, sl] + _unlo(we), 0.0)
                gth_v[r, sh] = jnp.maximum(gth_v[r, sh] + _unhi(we), 0.0)

    bufa = (src_a, dst_a, ebf_a, sem_ia)
    bufb = (src_b, dst_b, ebf_b, sem_ib)

    issue_in(0, *bufa)
    wait_in(0, *bufa)
    issue_g(src_a, gth_a, sem_ga)
    issue_in(1, *bufb)

    def pair(i, carry):
        t = 2 * i
        wait_g(src_a, gth_a, sem_ga)
        wait_in(t + 1, *bufb)
        issue_g(src_b, gth_b, sem_gb)
        relu(gth_a, ebf_a)
        sc_a = pltpu.async_copy(gth_a, aggr_sh.at[dst_a], sem_sa, add=True)
        wait_g(src_b, gth_b, sem_gb)
        sc_a.wait()
        issue_in(t + 2, *bufa)
        relu(gth_b, ebf_b)
        sc_b = pltpu.async_copy(gth_b, aggr_sh.at[dst_b], sem_sb, add=True)
        wait_in(t + 2, *bufa)
        issue_g(src_a, gth_a, sem_ga)
        sc_b.wait()

        @pl.when(t + 3 < CHUNKS_PER_W)
        def _pre():
            issue_in(t + 3, *bufb)
        return carry
    lax.fori_loop(0, (CHUNKS_PER_W - 1) // 2, pair, 0)

    wait_g(src_a, gth_a, sem_ga)
    relu(gth_a, ebf_a)
    pltpu.sync_copy(gth_a, aggr_sh.at[dst_a], add=True)

    plsc.subcore_barrier()
    for k in range(3):
        pltpu.sync_copy(aggr_sh.at[pl.ds(s * 624 + k * WB, WB), :],
                        out_hbm.at[c, pl.ds(s * 624 + k * WB, WB), :])

    @pl.when(s == NS - 1)
    def _otail():
        pltpu.sync_copy(aggr_sh.at[pl.ds(N - 16, 16), :],
                        out_hbm.at[c, pl.ds(N - 16, 16), :])


_sc_aggregate = pl.kernel(
    _sc_aggr_body,
    out_type=jax.ShapeDtypeStruct((NC, N, D), jnp.float32),
    mesh=plsc.VectorSubcoreMesh(core_axis_name="c", subcore_axis_name="s",
                                num_cores=NC, num_subcores=NS),
    scratch_types=[
        pltpu.VMEM((C,), jnp.int32),
        pltpu.VMEM((C,), jnp.int32),
        pltpu.VMEM((C, D // 2), jnp.int32),
        pltpu.VMEM((C, D), jnp.float32),
        pltpu.VMEM((C,), jnp.int32),
        pltpu.VMEM((C,), jnp.int32),
        pltpu.VMEM((C, D // 2), jnp.int32),
        pltpu.VMEM((C, D), jnp.float32),
        pltpu.VMEM_SHARED((N, D), jnp.float32),
        pltpu.SemaphoreType.DMA,
        pltpu.SemaphoreType.DMA,
        pltpu.SemaphoreType.DMA,
        pltpu.SemaphoreType.DMA,
        pltpu.SemaphoreType.DMA,
        pltpu.SemaphoreType.DMA,
    ],
)


def _mlp_body(h_ref, a_ref, w1_ref, b1_ref, w2_ref, b2_ref, sc_ref, out_ref,
              *, last):
    z = h_ref[...] * sc_ref[0, 0] + a_ref[0] + a_ref[1]
    z = jnp.maximum(
        jnp.dot(z, w1_ref[...], preferred_element_type=jnp.float32)
        + b1_ref[...], 0.0)
    z = jnp.dot(z, w2_ref[...], preferred_element_type=jnp.float32) + b2_ref[...]
    out_ref[...] = z if last else jnp.maximum(z, 0.0)


def _mlp(h, aggr, w1, b1, w2, b2, scale, last):
    NB = 1000
    return pl.pallas_call(
        functools.partial(_mlp_body, last=last),
        grid=(N // NB,),
        in_specs=[pl.BlockSpec((NB, D), lambda i: (i, 0)),
                  pl.BlockSpec((NC, NB, D), lambda i: (0, i, 0)),
                  pl.BlockSpec((D, D), lambda i: (0, 0)),
                  pl.BlockSpec((1, D), lambda i: (0, 0)),
                  pl.BlockSpec((D, D), lambda i: (0, 0)),
                  pl.BlockSpec((1, D), lambda i: (0, 0)),
                  pl.BlockSpec((1, 1), lambda i: (0, 0))],
        out_specs=pl.BlockSpec((NB, D), lambda i: (i, 0)),
        out_shape=jax.ShapeDtypeStruct((N, D), jnp.float32),
    )(h, aggr, w1, b1, w2, b2, scale)


def _pool_body(b_ref, h_ref, wp_ref, bp_ref, out_ref, pooled_acc, cnt_acc):
    i = pl.program_id(0)
    nblk = pl.num_programs(0)

    @pl.when(i == 0)
    def _init():
        pooled_acc[...] = jnp.zeros_like(pooled_acc)
        cnt_acc[...] = jnp.zeros_like(cnt_acc)

    nb = b_ref.shape[2]
    b = b_ref[0, 0, :]
    iota = lax.broadcasted_iota(jnp.int32, (nb, G), 1)
    p = (b[:, None] == iota).astype(jnp.float32)
    pooled_acc[...] += lax.dot_general(
        p, h_ref[...], (((0,), (0,)), ((), ())),
        preferred_element_type=jnp.float32)
    cnt_acc[...] += lax.dot_general(
        p, jnp.ones((nb, D), jnp.float32), (((0,), (0,)), ((), ())),
        preferred_element_type=jnp.float32)

    @pl.when(i == nblk - 1)
    def _fin():
        pooled = pooled_acc[...] / jnp.maximum(cnt_acc[...], 1.0)
        out_ref[...] = (jnp.dot(pooled, wp_ref[...],
                                preferred_element_type=jnp.float32)
                        + bp_ref[...])


def _pool_head(batch, h, wp, bp):
    NB = 1000
    return pl.pallas_call(
        _pool_body,
        grid=(N // NB,),
        in_specs=[pl.BlockSpec((1, 1, NB), lambda i: (i, 0, 0)),
                  pl.BlockSpec((NB, D), lambda i: (i, 0)),
                  pl.BlockSpec((D, T), lambda i: (0, 0)),
                  pl.BlockSpec((1, T), lambda i: (0, 0))],
        out_specs=pl.BlockSpec((G, T), lambda i: (0, 0)),
        out_shape=jax.ShapeDtypeStruct((G, T), jnp.float32),
        scratch_shapes=[pltpu.VMEM((G, D), jnp.float32),
                        pltpu.VMEM((G, D), jnp.float32)],
    )(batch.reshape(N // NB, 1, NB), h, wp, bp.reshape(1, T))


def kernel(x, edge_index, edge_attr, batch, We, W1, b1, W2, b2, eps, Wp, bp):
    src = edge_index[0]
    dst = edge_index[1]
    h = x
    for l in range(L):
        e = _edge_embed(edge_attr, We[l])
        aggr = _sc_aggregate(h, e, src, dst)
        h = _mlp(h, aggr, W1[l], b1[l].reshape(1, D), W2[l],
                 b2[l].reshape(1, D), (1.0 + eps[l]).reshape(1, 1),
                 last=(l == L - 1))
    return _pool_head(batch, h, Wp, bp)


# speedup vs baseline: 1.0920x; 1.0874x over previous
import functools

import jax
import jax.numpy as jnp
from jax import lax
from jax.experimental import pallas as pl
from jax.experimental.pallas import tpu as pltpu
from jax.experimental.pallas import tpu_sc as plsc

N = 10000
E = 320000
D = 128
DE = 16
L = 5
G = 128
T = 12

NC, NS = 2, 16
NW = NC * NS
C = 128
NCHUNKS = E // C
CPW = NCHUNKS // NW
ZR = 16
WB = 208


def _pack_rows(v):
    lo = v[:, :D // 2].astype(jnp.bfloat16).astype(jnp.float32)
    hi = v[:, D // 2:].astype(jnp.bfloat16).astype(jnp.float32)
    lo_u = jax.lax.bitcast_convert_type(lo, jnp.uint32) >> 16
    hi_u = jax.lax.bitcast_convert_type(hi, jnp.uint32) & jnp.uint32(0xFFFF0000)
    return jax.lax.bitcast_convert_type(lo_u | hi_u, jnp.int32)


def _edge_embed_body(ea_ref, w_ref, out_ref):
    e = jnp.maximum(
        jnp.dot(ea_ref[...], w_ref[...], preferred_element_type=jnp.float32), 0.0)
    out_ref[...] = _pack_rows(e)


def _edge_embed(edge_attr, w):
    EB = 8000
    return pl.pallas_call(
        _edge_embed_body,
        grid=(E // EB,),
        in_specs=[pl.BlockSpec((EB, DE), lambda i: (i, 0)),
                  pl.BlockSpec((DE, D), lambda i: (0, 0))],
        out_specs=pl.BlockSpec((EB, D // 2), lambda i: (i, 0)),
        out_shape=jax.ShapeDtypeStruct((E, D // 2), jnp.int32),
    )(edge_attr, w)


def _sc_aggr_body(h_hbm, e_hbm, src_hbm, dst_hbm, out_hbm,
                  src_a, dst_a, gth_a, src_b, dst_b, gth_b, ebf_v,
                  aggr_sh, sem_ia, sem_ib, sem_ga, sem_gb,
                  sem_sa, sem_sb, sem_e):
    c = lax.axis_index("c")
    s = lax.axis_index("s")
    w = s * NC + c
    nj = jnp.where(w < 2, CPW + 2, CPW)

    def cbase(t):
        jj = jnp.where(t < CPW, w + NW * t, NW * CPW + 2 * w + (t - CPW))
        return jj * C

    def zrow(r, carry):
        for q in range(D // 16):
            gth_a[r, pl.ds(q * 16, 16)] = jnp.zeros((16,), jnp.float32)
        return carry
    lax.fori_loop(0, ZR, zrow, 0)

    def zcopy(k, carry):
        pltpu.sync_copy(gth_a.at[pl.ds(0, ZR), :],
                        aggr_sh.at[pl.ds(s * 624 + k * ZR, ZR), :])
        return carry
    lax.fori_loop(0, 624 // ZR, zcopy, 0)

    @pl.when(s == NS - 1)
    def _ztail():
        pltpu.sync_copy(gth_a.at[pl.ds(0, ZR), :],
                        aggr_sh.at[pl.ds(N - 16, 16), :])
    plsc.subcore_barrier()

    def issue_in(t, src_v, dst_v, sem):
        base = cbase(t)
        pltpu.async_copy(src_hbm.at[pl.ds(base, C)], src_v, sem)
        pltpu.async_copy(dst_hbm.at[pl.ds(base, C)], dst_v, sem)

    def wait_in(t, src_v, dst_v, sem):
        base = cbase(t)
        pltpu.make_async_copy(src_hbm.at[pl.ds(base, C)], src_v, sem).wait()
        pltpu.make_async_copy(dst_hbm.at[pl.ds(base, C)], dst_v, sem).wait()

    def issue_e(t):
        pltpu.async_copy(e_hbm.at[pl.ds(cbase(t), C), :], ebf_v, sem_e)

    def wait_e(t):
        pltpu.make_async_copy(e_hbm.at[pl.ds(cbase(t), C), :], ebf_v,
                              sem_e).wait()

    def issue_g(src_v, gth_v, sem):
        return pltpu.async_copy(h_hbm.at[src_v], gth_v, sem)

    def wait_g(src_v, gth_v, sem):
        pltpu.make_async_copy(h_hbm.at[src_v], gth_v, sem).wait()

    def _unlo(wd):
        return jax.lax.bitcast_convert_type(wd << 16, jnp.float32)

    def _unhi(wd):
        return jax.lax.bitcast_convert_type(wd & jnp.int32(-65536), jnp.float32)

    def relu(gth_v):
        @plsc.parallel_loop(0, C, unroll=4)
        def _row(r):
            for k in range(D // 32):
                sl = pl.ds(k * 16, 16)
                sh = pl.ds((k + 4) * 16, 16)
                we = ebf_v[r, pl.ds(k * 16, 16)]
                gth_v[r, sl] = jnp.maximum(gth_v[r, sl] + _unlo(we), 0.0)
                gth_v[r, sh] = jnp.maximum(gth_v[r, sh] + _unhi(we), 0.0)

    bufa = (src_a, dst_a, sem_ia)
    bufb = (src_b, dst_b, sem_ib)

    issue_in(0, *bufa)
    issue_e(0)
    wait_in(0, *bufa)
    issue_g(src_a, gth_a, sem_ga)
    issue_in(1, *bufb)

    def pair(i, carry):
        t = 2 * i
        wait_g(src_a, gth_a, sem_ga)
        wait_in(t + 1, *bufb)
        issue_g(src_b, gth_b, sem_gb)
        wait_e(t)
        relu(gth_a)
        issue_e(t + 1)
        sc_a = pltpu.async_copy(gth_a, aggr_sh.at[dst_a], sem_sa, add=True)
        wait_g(src_b, gth_b, sem_gb)
        sc_a.wait()
        issue_in(t + 2, *bufa)
        wait_e(t + 1)
        relu(gth_b)
        issue_e(t + 2)
        sc_b = pltpu.async_copy(gth_b, aggr_sh.at[dst_b], sem_sb, add=True)
        wait_in(t + 2, *bufa)
        issue_g(src_a, gth_a, sem_ga)
        sc_b.wait()

        @pl.when(t + 3 < nj)
        def _pre():
            issue_in(t + 3, *bufb)
        return carry
    lax.fori_loop(0, (nj - 2) // 2, pair, 0)

    wait_g(src_a, gth_a, sem_ga)
    wait_in(nj - 1, *bufb)
    issue_g(src_b, gth_b, sem_gb)
    wait_e(nj - 2)
    relu(gth_a)
    issue_e(nj - 1)
    sc_a = pltpu.async_copy(gth_a, aggr_sh.at[dst_a], sem_sa, add=True)
    wait_g(src_b, gth_b, sem_gb)
    sc_a.wait()
    wait_e(nj - 1)
    relu(gth_b)
    pltpu.sync_copy(gth_b, aggr_sh.at[dst_b], add=True)

    plsc.subcore_barrier()
    for k in range(3):
        pltpu.sync_copy(aggr_sh.at[pl.ds(s * 624 + k * WB, WB), :],
                        out_hbm.at[c, pl.ds(s * 624 + k * WB, WB), :])

    @pl.when(s == NS - 1)
    def _otail():
        pltpu.sync_copy(aggr_sh.at[pl.ds(N - 16, 16), :],
                        out_hbm.at[c, pl.ds(N - 16, 16), :])


_sc_aggregate = pl.kernel(
    _sc_aggr_body,
    out_type=jax.ShapeDtypeStruct((NC, N, D), jnp.float32),
    mesh=plsc.VectorSubcoreMesh(core_axis_name="c", subcore_axis_name="s",
                                num_cores=NC, num_subcores=NS),
    scratch_types=[
        pltpu.VMEM((C,), jnp.int32),
        pltpu.VMEM((C,), jnp.int32),
        pltpu.VMEM((C, D), jnp.float32),
        pltpu.VMEM((C,), jnp.int32),
        pltpu.VMEM((C,), jnp.int32),
        pltpu.VMEM((C, D), jnp.float32),
        pltpu.VMEM((C, D // 2), jnp.int32),
        pltpu.VMEM_SHARED((N, D), jnp.float32),
        pltpu.SemaphoreType.DMA,
        pltpu.SemaphoreType.DMA,
        pltpu.SemaphoreType.DMA,
        pltpu.SemaphoreType.DMA,
        pltpu.SemaphoreType.DMA,
        pltpu.SemaphoreType.DMA,
        pltpu.SemaphoreType.DMA,
    ],
)


def _mlp_body(h_ref, a_ref, w1_ref, b1_ref, w2_ref, b2_ref, sc_ref, out_ref,
              *, last):
    z = h_ref[...] * sc_ref[0, 0] + a_ref[0] + a_ref[1]
    z = jnp.maximum(
        jnp.dot(z, w1_ref[...], preferred_element_type=jnp.float32)
        + b1_ref[...], 0.0)
    z = jnp.dot(z, w2_ref[...], preferred_element_type=jnp.float32) + b2_ref[...]
    out_ref[...] = z if last else jnp.maximum(z, 0.0)


def _mlp(h, aggr, w1, b1, w2, b2, scale, last):
    NB = 1000
    return pl.pallas_call(
        functools.partial(_mlp_body, last=last),
        grid=(N // NB,),
        in_specs=[pl.BlockSpec((NB, D), lambda i: (i, 0)),
                  pl.BlockSpec((NC, NB, D), lambda i: (0, i, 0)),
                  pl.BlockSpec((D, D), lambda i: (0, 0)),
                  pl.BlockSpec((1, D), lambda i: (0, 0)),
                  pl.BlockSpec((D, D), lambda i: (0, 0)),
                  pl.BlockSpec((1, D), lambda i: (0, 0)),
                  pl.BlockSpec((1, 1), lambda i: (0, 0))],
        out_specs=pl.BlockSpec((NB, D), lambda i: (i, 0)),
        out_shape=jax.ShapeDtypeStruct((N, D), jnp.float32),
    )(h, aggr, w1, b1, w2, b2, scale)


def _pool_body(b_ref, h_ref, wp_ref, bp_ref, out_ref, pooled_acc, cnt_acc):
    i = pl.program_id(0)
    nblk = pl.num_programs(0)

    @pl.when(i == 0)
    def _init():
        pooled_acc[...] = jnp.zeros_like(pooled_acc)
        cnt_acc[...] = jnp.zeros_like(cnt_acc)

    nb = b_ref.shape[2]
    b = b_ref[0, 0, :]
    iota = lax.broadcasted_iota(jnp.int32, (nb, G), 1)
    p = (b[:, None] == iota).astype(jnp.float32)
    pooled_acc[...] += lax.dot_general(
        p, h_ref[...], (((0,), (0,)), ((), ())),
        preferred_element_type=jnp.float32)
    cnt_acc[...] += lax.dot_general(
        p, jnp.ones((nb, D), jnp.float32), (((0,), (0,)), ((), ())),
        preferred_element_type=jnp.float32)

    @pl.when(i == nblk - 1)
    def _fin():
        pooled = pooled_acc[...] / jnp.maximum(cnt_acc[...], 1.0)
        out_ref[...] = (jnp.dot(pooled, wp_ref[...],
                                preferred_element_type=jnp.float32)
                        + bp_ref[...])


def _pool_head(batch, h, wp, bp):
    NB = 1000
    return pl.pallas_call(
        _pool_body,
        grid=(N // NB,),
        in_specs=[pl.BlockSpec((1, 1, NB), lambda i: (i, 0, 0)),
                  pl.BlockSpec((NB, D), lambda i: (i, 0)),
                  pl.BlockSpec((D, T), lambda i: (0, 0)),
                  pl.BlockSpec((1, T), lambda i: (0, 0))],
        out_specs=pl.BlockSpec((G, T), lambda i: (0, 0)),
        out_shape=jax.ShapeDtypeStruct((G, T), jnp.float32),
        scratch_shapes=[pltpu.VMEM((G, D), jnp.float32),
                        pltpu.VMEM((G, D), jnp.float32)],
    )(batch.reshape(N // NB, 1, NB), h, wp, bp.reshape(1, T))


def kernel(x, edge_index, edge_attr, batch, We, W1, b1, W2, b2, eps, Wp, bp):
    src = edge_index[0]
    dst = edge_index[1]
    h = x
    for l in range(L):
        e = _edge_embed(edge_attr, We[l])
        aggr = _sc_aggregate(h, e, src, dst)
        h = _mlp(h, aggr, W1[l], b1[l].reshape(1, D), W2[l],
                 b2[l].reshape(1, D), (1.0 + eps[l]).reshape(1, 1),
                 last=(l == L - 1))
    return _pool_head(batch, h, Wp, bp)


# speedup vs baseline: 1.0943x; 1.0021x over previous
import functools

import jax
import jax.numpy as jnp
from jax import lax
from jax.experimental import pallas as pl
from jax.experimental.pallas import tpu as pltpu
from jax.experimental.pallas import tpu_sc as plsc

N = 10000
E = 320000
D = 128
DE = 16
L = 5
G = 128
T = 12

NC, NS = 2, 16
NW = NC * NS
C = 128
NCHUNKS = E // C
CPW = NCHUNKS // NW
ZR = 16
WB = 208


def _pack_rows(v):
    lo = v[:, :D // 2].astype(jnp.bfloat16).astype(jnp.float32)
    hi = v[:, D // 2:].astype(jnp.bfloat16).astype(jnp.float32)
    lo_u = jax.lax.bitcast_convert_type(lo, jnp.uint32) >> 16
    hi_u = jax.lax.bitcast_convert_type(hi, jnp.uint32) & jnp.uint32(0xFFFF0000)
    return jax.lax.bitcast_convert_type(lo_u | hi_u, jnp.int32)


def _edge_embed_body(ea_ref, w_ref, out_ref):
    e = jnp.maximum(
        jnp.dot(ea_ref[...], w_ref[...], preferred_element_type=jnp.float32), 0.0)
    out_ref[...] = _pack_rows(e)


def _edge_embed(edge_attr, w):
    EB = 8000
    return pl.pallas_call(
        _edge_embed_body,
        grid=(E // EB,),
        in_specs=[pl.BlockSpec((EB, DE), lambda i: (i, 0)),
                  pl.BlockSpec((DE, D), lambda i: (0, 0))],
        out_specs=pl.BlockSpec((EB, D // 2), lambda i: (i, 0)),
        out_shape=jax.ShapeDtypeStruct((E, D // 2), jnp.int32),
    )(edge_attr, w)


def _sc_aggr_body(h_hbm, e_hbm, src_hbm, dst_hbm, out_hbm,
                  src_a, dst_a, gth_a, src_b, dst_b, gth_b, ebf_v,
                  aggr_sh, sem_ia, sem_ib, sem_ga, sem_gb,
                  sem_sa, sem_sb, sem_e):
    c = lax.axis_index("c")
    s = lax.axis_index("s")
    w = s * NC + c
    nj = jnp.where(w < 2, CPW + 2, CPW)

    def cbase(t):
        jj = jnp.where(t < CPW, w + NW * t, NW * CPW + 2 * w + (t - CPW))
        return jj * C

    def zrow(r, carry):
        for q in range(D // 16):
            gth_a[r, pl.ds(q * 16, 16)] = jnp.zeros((16,), jnp.float32)
        return carry
    lax.fori_loop(0, 104, zrow, 0)

    zblk = gth_a.at[pl.ds(0, 104), :]
    for k in range(6):
        pltpu.async_copy(zblk, aggr_sh.at[pl.ds(s * 624 + k * 104, 104), :],
                         sem_e)

    @pl.when(s == NS - 1)
    def _ztail():
        pltpu.async_copy(gth_a.at[pl.ds(0, 16), :],
                         aggr_sh.at[pl.ds(N - 16, 16), :], sem_e)
    for k in range(6):
        pltpu.make_async_copy(zblk,
                              aggr_sh.at[pl.ds(s * 624 + k * 104, 104), :],
                              sem_e).wait()

    @pl.when(s == NS - 1)
    def _ztail2():
        pltpu.make_async_copy(gth_a.at[pl.ds(0, 16), :],
                              aggr_sh.at[pl.ds(N - 16, 16), :], sem_e).wait()
    plsc.subcore_barrier()

    def issue_in(t, src_v, dst_v, sem):
        base = cbase(t)
        pltpu.async_copy(src_hbm.at[pl.ds(base, C)], src_v, sem)
        pltpu.async_copy(dst_hbm.at[pl.ds(base, C)], dst_v, sem)

    def wait_in(t, src_v, dst_v, sem):
        base = cbase(t)
        pltpu.make_async_copy(src_hbm.at[pl.ds(base, C)], src_v, sem).wait()
        pltpu.make_async_copy(dst_hbm.at[pl.ds(base, C)], dst_v, sem).wait()

    def issue_e(t):
        pltpu.async_copy(e_hbm.at[pl.ds(cbase(t), C), :], ebf_v, sem_e)

    def wait_e(t):
        pltpu.make_async_copy(e_hbm.at[pl.ds(cbase(t), C), :], ebf_v,
                              sem_e).wait()

    def issue_g(src_v, gth_v, sem):
        return pltpu.async_copy(h_hbm.at[src_v], gth_v, sem)

    def wait_g(src_v, gth_v, sem):
        pltpu.make_async_copy(h_hbm.at[src_v], gth_v, sem).wait()

    def _unlo(wd):
        return jax.lax.bitcast_convert_type(wd << 16, jnp.float32)

    def _unhi(wd):
        return jax.lax.bitcast_convert_type(wd & jnp.int32(-65536), jnp.float32)

    def relu(gth_v):
        @plsc.parallel_loop(0, C, unroll=4)
        def _row(r):
            for k in range(D // 32):
                sl = pl.ds(k * 16, 16)
                sh = pl.ds((k + 4) * 16, 16)
                we = ebf_v[r, pl.ds(k * 16, 16)]
                gth_v[r, sl] = jnp.maximum(gth_v[r, sl] + _unlo(we), 0.0)
                gth_v[r, sh] = jnp.maximum(gth_v[r, sh] + _unhi(we), 0.0)

    bufa = (src_a, dst_a, sem_ia)
    bufb = (src_b, dst_b, sem_ib)

    issue_in(0, *bufa)
    issue_e(0)
    wait_in(0, *bufa)
    issue_g(src_a, gth_a, sem_ga)
    issue_in(1, *bufb)

    def pair(i, carry):
        t = 2 * i
        wait_g(src_a, gth_a, sem_ga)
        wait_in(t + 1, *bufb)
        issue_g(src_b, gth_b, sem_gb)
        wait_e(t)
        relu(gth_a)
        issue_e(t + 1)
        sc_a = pltpu.async_copy(gth_a, aggr_sh.at[dst_a], sem_sa, add=True)
        wait_g(src_b, gth_b, sem_gb)
        sc_a.wait()
        issue_in(t + 2, *bufa)
        wait_e(t + 1)
        relu(gth_b)
        issue_e(t + 2)
        sc_b = pltpu.async_copy(gth_b, aggr_sh.at[dst_b], sem_sb, add=True)
        wait_in(t + 2, *bufa)
        issue_g(src_a, gth_a, sem_ga)
        sc_b.wait()

        @pl.when(t + 3 < nj)
        def _pre():
            issue_in(t + 3, *bufb)
        return carry
    lax.fori_loop(0, (nj - 2) // 2, pair, 0)

    wait_g(src_a, gth_a, sem_ga)
    wait_in(nj - 1, *bufb)
    issue_g(src_b, gth_b, sem_gb)
    wait_e(nj - 2)
    relu(gth_a)
    issue_e(nj - 1)
    sc_a = pltpu.async_copy(gth_a, aggr_sh.at[dst_a], sem_sa, add=True)
    wait_g(src_b, gth_b, sem_gb)
    sc_a.wait()
    wait_e(nj - 1)
    relu(gth_b)
    pltpu.sync_copy(gth_b, aggr_sh.at[dst_b], add=True)

    plsc.subcore_barrier()
    for k in range(3):
        pltpu.async_copy(aggr_sh.at[pl.ds(s * 624 + k * WB, WB), :],
                         out_hbm.at[c, pl.ds(s * 624 + k * WB, WB), :], sem_e)

    @pl.when(s == NS - 1)
    def _otail():
        pltpu.async_copy(aggr_sh.at[pl.ds(N - 16, 16), :],
                         out_hbm.at[c, pl.ds(N - 16, 16), :], sem_e)
    for k in range(3):
        pltpu.make_async_copy(aggr_sh.at[pl.ds(s * 624 + k * WB, WB), :],
                              out_hbm.at[c, pl.ds(s * 624 + k * WB, WB), :],
                              sem_e).wait()

    @pl.when(s == NS - 1)
    def _otail2():
        pltpu.make_async_copy(aggr_sh.at[pl.ds(N - 16, 16), :],
                              out_hbm.at[c, pl.ds(N - 16, 16), :],
                              sem_e).wait()


_sc_aggregate = pl.kernel(
    _sc_aggr_body,
    out_type=jax.ShapeDtypeStruct((NC, N, D), jnp.float32),
    mesh=plsc.VectorSubcoreMesh(core_axis_name="c", subcore_axis_name="s",
                                num_cores=NC, num_subcores=NS),
    scratch_types=[
        pltpu.VMEM((C,), jnp.int32),
        pltpu.VMEM((C,), jnp.int32),
        pltpu.VMEM((C, D), jnp.float32),
        pltpu.VMEM((C,), jnp.int32),
        pltpu.VMEM((C,), jnp.int32),
        pltpu.VMEM((C, D), jnp.float32),
        pltpu.VMEM((C, D // 2), jnp.int32),
        pltpu.VMEM_SHARED((N, D), jnp.float32),
        pltpu.SemaphoreType.DMA,
        pltpu.SemaphoreType.DMA,
        pltpu.SemaphoreType.DMA,
        pltpu.SemaphoreType.DMA,
        pltpu.SemaphoreType.DMA,
        pltpu.SemaphoreType.DMA,
        pltpu.SemaphoreType.DMA,
    ],
)


def _mlp_body(h_ref, a_ref, w1_ref, b1_ref, w2_ref, b2_ref, sc_ref, out_ref,
              *, last):
    z = h_ref[...] * sc_ref[0, 0] + a_ref[0] + a_ref[1]
    z = jnp.maximum(
        jnp.dot(z, w1_ref[...], preferred_element_type=jnp.float32)
        + b1_ref[...], 0.0)
    z = jnp.dot(z, w2_ref[...], preferred_element_type=jnp.float32) + b2_ref[...]
    out_ref[...] = z if last else jnp.maximum(z, 0.0)


def _mlp(h, aggr, w1, b1, w2, b2, scale, last):
    NB = 1000
    return pl.pallas_call(
        functools.partial(_mlp_body, last=last),
        grid=(N // NB,),
        in_specs=[pl.BlockSpec((NB, D), lambda i: (i, 0)),
                  pl.BlockSpec((NC, NB, D), lambda i: (0, i, 0)),
                  pl.BlockSpec((D, D), lambda i: (0, 0)),
                  pl.BlockSpec((1, D), lambda i: (0, 0)),
                  pl.BlockSpec((D, D), lambda i: (0, 0)),
                  pl.BlockSpec((1, D), lambda i: (0, 0)),
                  pl.BlockSpec((1, 1), lambda i: (0, 0))],
        out_specs=pl.BlockSpec((NB, D), lambda i: (i, 0)),
        out_shape=jax.ShapeDtypeStruct((N, D), jnp.float32),
    )(h, aggr, w1, b1, w2, b2, scale)


def _pool_body(b_ref, h_ref, wp_ref, bp_ref, out_ref, pooled_acc, cnt_acc):
    i = pl.program_id(0)
    nblk = pl.num_programs(0)

    @pl.when(i == 0)
    def _init():
        pooled_acc[...] = jnp.zeros_like(pooled_acc)
        cnt_acc[...] = jnp.zeros_like(cnt_acc)

    nb = b_ref.shape[2]
    b = b_ref[0, 0, :]
    iota = lax.broadcasted_iota(jnp.int32, (nb, G), 1)
    p = (b[:, None] == iota).astype(jnp.float32)
    pooled_acc[...] += lax.dot_general(
        p, h_ref[...], (((0,), (0,)), ((), ())),
        preferred_element_type=jnp.float32)
    cnt_acc[...] += lax.dot_general(
        p, jnp.ones((nb, D), jnp.float32), (((0,), (0,)), ((), ())),
        preferred_element_type=jnp.float32)

    @pl.when(i == nblk - 1)
    def _fin():
        pooled = pooled_acc[...] / jnp.maximum(cnt_acc[...], 1.0)
        out_ref[...] = (jnp.dot(pooled, wp_ref[...],
                                preferred_element_type=jnp.float32)
                        + bp_ref[...])


def _pool_head(batch, h, wp, bp):
    NB = 1000
    return pl.pallas_call(
        _pool_body,
        grid=(N // NB,),
        in_specs=[pl.BlockSpec((1, 1, NB), lambda i: (i, 0, 0)),
                  pl.BlockSpec((NB, D), lambda i: (i, 0)),
                  pl.BlockSpec((D, T), lambda i: (0, 0)),
                  pl.BlockSpec((1, T), lambda i: (0, 0))],
        out_specs=pl.BlockSpec((G, T), lambda i: (0, 0)),
        out_shape=jax.ShapeDtypeStruct((G, T), jnp.float32),
        scratch_shapes=[pltpu.VMEM((G, D), jnp.float32),
                        pltpu.VMEM((G, D), jnp.float32)],
    )(batch.reshape(N // NB, 1, NB), h, wp, bp.reshape(1, T))


def kernel(x, edge_index, edge_attr, batch, We, W1, b1, W2, b2, eps, Wp, bp):
    src = edge_index[0]
    dst = edge_index[1]
    h = x
    for l in range(L):
        e = _edge_embed(edge_attr, We[l])
        aggr = _sc_aggregate(h, e, src, dst)
        h = _mlp(h, aggr, W1[l], b1[l].reshape(1, D), W2[l],
                 b2[l].reshape(1, D), (1.0 + eps[l]).reshape(1, 1),
                 last=(l == L - 1))
    return _pool_head(batch, h, Wp, bp)
